# Initial kernel scaffold; baseline (speedup 1.0000x reference)
import jax
import jax.numpy as jnp
from jax.experimental import pallas as pl


def kernel(boxes, scores):
    raise NotImplementedError("write your pallas kernel here")


# speedup vs baseline: 42.3141x; 42.3141x over previous
import jax
import jax.numpy as jnp
from jax.experimental import pallas as pl
from jax.experimental.pallas import tpu as pltpu

_N = 5000
_NP = 5120
_B = 512
_NB = _NP // _B
_T = 0.5


def _nms_kernel(drows_ref, dcols_ref, out_ref,
                rankc_ref, rankr_ref, srows_ref, scols_ref, sup_ref):
    f32 = jnp.float32
    sub_b = jax.lax.broadcasted_iota(jnp.int32, (_B, _B), 0)
    lane_b = jax.lax.broadcasted_iota(jnp.int32, (_B, _B), 1)
    diag_b = jnp.where(sub_b == lane_b, 1.0, 0.0).astype(f32)

    def row2col(v):
        return jnp.sum(diag_b * v, axis=1, keepdims=True)

    def col2row(v):
        return jnp.sum(diag_b * v, axis=0, keepdims=True)

    s_row = drows_ref[4:5, :]
    sub_np = jax.lax.broadcasted_iota(jnp.int32, (_B, _NP), 0)
    lane_np = jax.lax.broadcasted_iota(jnp.int32, (_B, _NP), 1)

    def rank_body(bi, carry):
        base = bi * _B
        sb = dcols_ref[pl.ds(base, _B), 4:5]
        gt = s_row > sb
        eq = (s_row == sb) & (lane_np < (base + sub_np))
        cnt = jnp.sum(jnp.where(gt | eq, 1.0, 0.0).astype(f32),
                      axis=1, keepdims=True)
        rankc_ref[pl.ds(base, _B), :] = cnt
        rankr_ref[0:1, pl.ds(base, _B)] = col2row(cnt)
        return carry

    jax.lax.fori_loop(0, _NB, rank_body, 0)

    lane_bc = jax.lax.broadcasted_iota(jnp.int32, (_NP, _B), 1)

    def perm_body(bi, carry):
        base = bi * _B
        onehot_t = jnp.where(rankc_ref[:, :] == (base + lane_bc).astype(f32),
                             1.0, 0.0).astype(f32)
        srows_ref[:, pl.ds(base, _B)] = jnp.dot(
            drows_ref[:, :], onehot_t,
            preferred_element_type=f32,
            precision=jax.lax.Precision.HIGHEST)
        onehot = jnp.where(rankr_ref[0:1, :] == (base + sub_np).astype(f32),
                           1.0, 0.0).astype(f32)
        scols_ref[pl.ds(base, _B), :] = jnp.dot(
            onehot, dcols_ref[:, :],
            preferred_element_type=f32,
            precision=jax.lax.Precision.HIGHEST)
        return carry

    jax.lax.fori_loop(0, _NB, perm_body, 0)

    sup_ref[:, :] = jnp.zeros((1, _NP), f32)

    def nms_body(bi, carry):
        base = bi * _B
        bx1 = scols_ref[pl.ds(base, _B), 0:1]
        by1 = scols_ref[pl.ds(base, _B), 1:2]
        bx2 = scols_ref[pl.ds(base, _B), 2:3]
        by2 = scols_ref[pl.ds(base, _B), 3:4]
        barea = (bx2 - bx1) * (by2 - by1)

        rx1 = srows_ref[0:1, pl.ds(base, _B)]
        ry1 = srows_ref[1:2, pl.ds(base, _B)]
        rx2 = srows_ref[2:3, pl.ds(base, _B)]
        ry2 = srows_ref[3:4, pl.ds(base, _B)]
        rarea = (rx2 - rx1) * (ry2 - ry1)

        wx = jnp.maximum(jnp.minimum(bx2, rx2) - jnp.maximum(bx1, rx1), 0.0)
        wy = jnp.maximum(jnp.minimum(by2, ry2) - jnp.maximum(by1, ry1), 0.0)
        inter = wx * wy
        iou = inter / jnp.maximum(barea + rarea - inter, 1e-9)
        m = jnp.where((iou > _T) & (sub_b < lane_b), 1.0, 0.0).astype(f32)

        ext_row = 1.0 - sup_ref[0:1, pl.ds(base, _B)]
        ext_col = row2col(ext_row)

        def fp_cond(c):
            return c[1]

        def fp_body(c):
            kc, _ = c
            s = jnp.max(m * kc, axis=0, keepdims=True)
            kr = ext_row * (1.0 - s)
            kc2 = row2col(kr)
            changed = jnp.max(jnp.abs(kc2 - kc)) > 0.0
            return (kc2, changed)

        keep_col, _ = jax.lax.while_loop(
            fp_cond, fp_body, (ext_col, jnp.array(True)))

        gx1 = srows_ref[0:1, :]
        gy1 = srows_ref[1:2, :]
        gx2 = srows_ref[2:3, :]
        gy2 = srows_ref[3:4, :]
        garea = (gx2 - gx1) * (gy2 - gy1)
        cwx = jnp.maximum(jnp.minimum(bx2, gx2) - jnp.maximum(bx1, gx1), 0.0)
        cwy = jnp.maximum(jnp.minimum(by2, gy2) - jnp.maximum(by1, gy1), 0.0)
        cinter = cwx * cwy
        ciou = cinter / jnp.maximum(barea + garea - cinter, 1e-9)
        hit = jnp.where(ciou > _T, 1.0, 0.0).astype(f32) * keep_col
        sup_new = jnp.max(hit, axis=0, keepdims=True)
        sup_ref[:, :] = jnp.maximum(sup_ref[:, :], sup_new)

        out_ref[pl.ds(base, _B), :] = scols_ref[pl.ds(base, _B), :] * keep_col
        return carry

    jax.lax.fori_loop(0, _NB, nms_body, 0)


def kernel(boxes, scores):
    b = jnp.zeros((_NP, 4), jnp.float32).at[:_N].set(boxes.astype(jnp.float32))
    s = jnp.full((_NP,), -1.0, jnp.float32).at[:_N].set(
        scores.astype(jnp.float32))
    dcols = jnp.concatenate(
        [b, s[:, None], jnp.zeros((_NP, 3), jnp.float32)], axis=1)
    drows = dcols.T
    out = pl.pallas_call(
        _nms_kernel,
        out_shape=jax.ShapeDtypeStruct((_NP, 8), jnp.float32),
        scratch_shapes=[
            pltpu.VMEM((_NP, 1), jnp.float32),
            pltpu.VMEM((1, _NP), jnp.float32),
            pltpu.VMEM((8, _NP), jnp.float32),
            pltpu.VMEM((_NP, 8), jnp.float32),
            pltpu.VMEM((1, _NP), jnp.float32),
        ],
    )(drows, dcols)
    return out[:_N, :5]


# speedup vs baseline: 82.6936x; 1.9543x over previous
import jax
import jax.numpy as jnp
from jax.experimental import pallas as pl
from jax.experimental.pallas import tpu as pltpu

_N = 5000
_NP = 5120
_B = 512
_NB = _NP // _B
_T = 0.5


def _nms_kernel(drows_ref, scol_ref, out_ref, rankc_ref, srows_ref, sup_ref):
    f32 = jnp.float32
    sub_b = jax.lax.broadcasted_iota(jnp.int32, (_B, _B), 0)
    lane_b = jax.lax.broadcasted_iota(jnp.int32, (_B, _B), 1)
    diag_b = jnp.where(sub_b == lane_b, 1.0, 0.0).astype(f32)

    def row2col(v):
        return jnp.sum(diag_b * v, axis=1, keepdims=True)

    def col2row(v):
        return jnp.sum(diag_b * v, axis=0, keepdims=True)

    s_row = drows_ref[4:5, :]
    sub_np = jax.lax.broadcasted_iota(jnp.int32, (_B, _NP), 0)
    lane_np = jax.lax.broadcasted_iota(jnp.int32, (_B, _NP), 1)

    def rank_body(bi, carry):
        base = bi * _B
        sb = scol_ref[pl.ds(base, _B), :]
        gt = s_row > sb
        eq = (s_row == sb) & (lane_np < (base + sub_np))
        cnt = jnp.sum(jnp.where(gt | eq, 1.0, 0.0).astype(f32),
                      axis=1, keepdims=True)
        rankc_ref[pl.ds(base, _B), :] = cnt
        return carry

    jax.lax.fori_loop(0, _NB, rank_body, 0)

    lane_bc = jax.lax.broadcasted_iota(jnp.int32, (_NP, _B), 1)

    def perm_body(bi, carry):
        base = bi * _B
        onehot_t = jnp.where(rankc_ref[:, :] == (base + lane_bc).astype(f32),
                             1.0, 0.0).astype(f32)
        srows_ref[:, pl.ds(base, _B)] = jnp.dot(
            drows_ref[:, :], onehot_t,
            preferred_element_type=f32,
            precision=jax.lax.Precision.HIGHEST)
        return carry

    jax.lax.fori_loop(0, _NB, perm_body, 0)

    sup_ref[:, :] = jnp.zeros((1, _NP), f32)

    for bi in range(_NB):
        base = bi * _B
        rx1 = srows_ref[0:1, base:base + _B]
        ry1 = srows_ref[1:2, base:base + _B]
        rx2 = srows_ref[2:3, base:base + _B]
        ry2 = srows_ref[3:4, base:base + _B]
        rarea = (rx2 - rx1) * (ry2 - ry1)

        bx1 = row2col(rx1)
        by1 = row2col(ry1)
        bx2 = row2col(rx2)
        by2 = row2col(ry2)
        barea = (bx2 - bx1) * (by2 - by1)

        wx = jnp.maximum(jnp.minimum(bx2, rx2) - jnp.maximum(bx1, rx1), 0.0)
        wy = jnp.maximum(jnp.minimum(by2, ry2) - jnp.maximum(by1, ry1), 0.0)
        inter = wx * wy
        iou = inter / jnp.maximum(barea + rarea - inter, 1e-9)
        m = jnp.where((iou > _T) & (sub_b < lane_b), 1.0, 0.0).astype(f32)

        ext_row = 1.0 - sup_ref[0:1, base:base + _B]
        ext_col = row2col(ext_row)

        def fp_cond(c):
            return c[1]

        def fp_body(c):
            kc, _ = c
            s = jnp.max(m * kc, axis=0, keepdims=True)
            kr = ext_row * (1.0 - s)
            kc2 = row2col(kr)
            changed = jnp.max(jnp.abs(kc2 - kc)) > 0.0
            return (kc2, changed)

        keep_col, _ = jax.lax.while_loop(
            fp_cond, fp_body, (ext_col, jnp.array(True)))
        keep_row = col2row(keep_col)

        out_ref[:, base:base + _B] = srows_ref[:, base:base + _B] * keep_row

        rest = _NP - base - _B
        if rest == 0:
            continue
        lo = base + _B
        gx1 = srows_ref[0:1, lo:]
        gy1 = srows_ref[1:2, lo:]
        gx2 = srows_ref[2:3, lo:]
        gy2 = srows_ref[3:4, lo:]
        garea = (gx2 - gx1) * (gy2 - gy1)
        cwx = jnp.maximum(jnp.minimum(bx2, gx2) - jnp.maximum(bx1, gx1), 0.0)
        cwy = jnp.maximum(jnp.minimum(by2, gy2) - jnp.maximum(by1, gy1), 0.0)
        cinter = cwx * cwy
        ciou = cinter / jnp.maximum(barea + garea - cinter, 1e-9)
        hit = jnp.where(ciou > _T, 1.0, 0.0).astype(f32) * keep_col
        sup_new = jnp.max(hit, axis=0, keepdims=True)
        sup_ref[0:1, lo:] = jnp.maximum(sup_ref[0:1, lo:], sup_new)


def kernel(boxes, scores):
    b = jnp.zeros((_NP, 4), jnp.float32).at[:_N].set(boxes.astype(jnp.float32))
    s = jnp.full((_NP,), -1.0, jnp.float32).at[:_N].set(
        scores.astype(jnp.float32))
    drows = jnp.concatenate(
        [b, s[:, None], jnp.zeros((_NP, 3), jnp.float32)], axis=1).T
    scol = s[:, None]
    out = pl.pallas_call(
        _nms_kernel,
        out_shape=jax.ShapeDtypeStruct((8, _NP), jnp.float32),
        scratch_shapes=[
            pltpu.VMEM((_NP, 1), jnp.float32),
            pltpu.VMEM((8, _NP), jnp.float32),
            pltpu.VMEM((1, _NP), jnp.float32),
        ],
    )(drows, scol)
    return out.T[:_N, :5]


# speedup vs baseline: 125.3179x; 1.5154x over previous
import jax
import jax.numpy as jnp
from jax.experimental import pallas as pl
from jax.experimental.pallas import tpu as pltpu

_N = 5000
_NP = 5120
_B = 512
_NB = _NP // _B
_T = 0.5


def _nms_kernel(drows_ref, scol_ref, out_ref, rankc_ref, srows_ref, sup_ref):
    f32 = jnp.float32
    sub_b = jax.lax.broadcasted_iota(jnp.int32, (_B, _B), 0)
    lane_b = jax.lax.broadcasted_iota(jnp.int32, (_B, _B), 1)
    diag_b = jnp.where(sub_b == lane_b, 1.0, 0.0).astype(f32)

    def row2col(v):
        return jnp.sum(diag_b * v, axis=1, keepdims=True)

    def col2row(v):
        return jnp.sum(diag_b * v, axis=0, keepdims=True)

    s_row = drows_ref[4:5, :]
    sub_np = jax.lax.broadcasted_iota(jnp.int32, (_B, _NP), 0)
    lane_np = jax.lax.broadcasted_iota(jnp.int32, (_B, _NP), 1)
    d_np = lane_np - sub_np

    def rank_body(bi, carry):
        base = bi * _B
        sb = scol_ref[pl.ds(base, _B), :]
        gt = s_row > sb
        eq = (s_row == sb) & (d_np < base)
        cnt = jnp.sum(jnp.where(gt | eq, 1.0, 0.0).astype(f32),
                      axis=1, keepdims=True)
        rankc_ref[pl.ds(base, _B), :] = cnt
        return carry

    jax.lax.fori_loop(0, _NB, rank_body, 0)

    lane_bc = jax.lax.broadcasted_iota(jnp.int32, (_NP, _B), 1)

    def perm_body(bi, carry):
        base = bi * _B
        onehot_t = jnp.where(rankc_ref[:, :] == (base + lane_bc).astype(f32),
                             1.0, 0.0).astype(f32)
        srows_ref[:, pl.ds(base, _B)] = jnp.dot(
            drows_ref[:, :], onehot_t,
            preferred_element_type=f32)
        return carry

    jax.lax.fori_loop(0, _NB, perm_body, 0)

    sup_ref[:, :] = jnp.zeros((1, _NP), f32)

    for bi in range(_NB):
        base = bi * _B
        rx1 = srows_ref[0:1, base:base + _B]
        ry1 = srows_ref[1:2, base:base + _B]
        rx2 = srows_ref[2:3, base:base + _B]
        ry2 = srows_ref[3:4, base:base + _B]
        rarea = (rx2 - rx1) * (ry2 - ry1)

        bx1 = row2col(rx1)
        by1 = row2col(ry1)
        bx2 = row2col(rx2)
        by2 = row2col(ry2)
        barea = (bx2 - bx1) * (by2 - by1)

        wx = jnp.maximum(jnp.minimum(bx2, rx2) - jnp.maximum(bx1, rx1), 0.0)
        wy = jnp.maximum(jnp.minimum(by2, ry2) - jnp.maximum(by1, ry1), 0.0)
        inter = wx * wy
        iou = inter / jnp.maximum(barea + rarea - inter, 1e-9)
        m = jnp.where((iou > _T) & (sub_b < lane_b), 1.0, 0.0).astype(f32)

        ext_row = 1.0 - sup_ref[0:1, base:base + _B]
        ext_col = row2col(ext_row)

        def fp_cond(c):
            return c[2]

        def fp_body(c):
            kc, kr, _ = c
            s = jnp.max(m * kc, axis=0, keepdims=True)
            kr2 = ext_row * (1.0 - s)
            kc2 = row2col(kr2)
            changed = jnp.max(jnp.abs(kr2 - kr)) > 0.0
            return (kc2, kr2, changed)

        keep_col, keep_row, _ = jax.lax.while_loop(
            fp_cond, fp_body, (ext_col, ext_row, jnp.array(True)))

        out_ref[:, base:base + _B] = srows_ref[:, base:base + _B] * keep_row

        rest = _NP - base - _B
        if rest == 0:
            continue
        lo = base + _B
        gx1 = srows_ref[0:1, lo:]
        gy1 = srows_ref[1:2, lo:]
        gx2 = srows_ref[2:3, lo:]
        gy2 = srows_ref[3:4, lo:]
        garea = (gx2 - gx1) * (gy2 - gy1)
        cwx = jnp.maximum(jnp.minimum(bx2, gx2) - jnp.maximum(bx1, gx1), 0.0)
        cwy = jnp.maximum(jnp.minimum(by2, gy2) - jnp.maximum(by1, gy1), 0.0)
        cinter = cwx * cwy
        ciou = cinter / jnp.maximum(barea + garea - cinter, 1e-9)
        hit = jnp.where(ciou > _T, keep_col, 0.0)
        sup_new = jnp.max(hit, axis=0, keepdims=True)
        sup_ref[0:1, lo:] = jnp.maximum(sup_ref[0:1, lo:], sup_new)


def kernel(boxes, scores):
    b = jnp.zeros((_NP, 4), jnp.float32).at[:_N].set(boxes.astype(jnp.float32))
    s = jnp.full((_NP,), -1.0, jnp.float32).at[:_N].set(
        scores.astype(jnp.float32))
    drows = jnp.concatenate(
        [b, s[:, None], jnp.zeros((_NP, 3), jnp.float32)], axis=1).T
    scol = s[:, None]
    out = pl.pallas_call(
        _nms_kernel,
        out_shape=jax.ShapeDtypeStruct((8, _NP), jnp.float32),
        scratch_shapes=[
            pltpu.VMEM((_NP, 1), jnp.float32),
            pltpu.VMEM((8, _NP), jnp.float32),
            pltpu.VMEM((1, _NP), jnp.float32),
        ],
    )(drows, scol)
    return out.T[:_N, :5]
